# Initial kernel scaffold; baseline (speedup 1.0000x reference)
import jax
import jax.numpy as jnp
from jax import lax
from jax.experimental import pallas as pl
from jax.experimental.pallas import tpu as pltpu
from jax.experimental.pallas import tpu_sc as plsc
import functools

N_NODES = 10000
N_EDGES = 320000
D = 128
NC = 2
NS = 16
NW = NC * NS
EPT = N_EDGES // NW
K = 80
NCHUNK = EPT // K
ROWS_PER_TILE = N_NODES // NS
SCALE = 1.0 / 33.0

_mesh = plsc.VectorSubcoreMesh(core_axis_name="c", subcore_axis_name="s")


@functools.partial(
    pl.kernel,
    out_type=jax.ShapeDtypeStruct((NC, N_NODES, D), jnp.float32),
    mesh=_mesh,
    scratch_types=[
        pltpu.VMEM((NCHUNK, K), jnp.int32),
        pltpu.VMEM((NCHUNK, K), jnp.int32),
        pltpu.VMEM((K, D), jnp.float32),
        pltpu.VMEM((NCHUNK, D), jnp.float32),
        pltpu.VMEM_SHARED((N_NODES, D), jnp.float32),
    ],
)
def _sc_aggregate(x_hbm, src_hbm, dst_hbm, out_hbm, src_v, dst_v, rows_v,
                  zbuf, agg_sp):
    c = lax.axis_index("c")
    s = lax.axis_index("s")
    wid = c * NS + s

    def zero_body(i, _):
        zbuf[i // 8, pl.ds((i % 8) * 16, 16)] = jnp.zeros((16,), jnp.float32)
        return 0
    lax.fori_loop(0, NCHUNK * (D // 16), zero_body, 0)

    def zcopy_body(k, _):
        pltpu.sync_copy(zbuf, agg_sp.at[pl.ds(s * ROWS_PER_TILE + k * NCHUNK,
                                              NCHUNK)])
        return 0
    lax.fori_loop(0, ROWS_PER_TILE // NCHUNK, zcopy_body, 0)

    pltpu.sync_copy(src_hbm.at[pl.ds(wid * NCHUNK, NCHUNK)], src_v)
    pltpu.sync_copy(dst_hbm.at[pl.ds(wid * NCHUNK, NCHUNK)], dst_v)

    plsc.subcore_barrier()

    def edge_body(j, _):
        pltpu.sync_copy(x_hbm.at[src_v.at[j]], rows_v)
        pltpu.sync_copy(rows_v, agg_sp.at[dst_v.at[j]], add=True)
        return 0
    lax.fori_loop(0, NCHUNK, edge_body, 0)

    plsc.subcore_barrier()

    pltpu.sync_copy(agg_sp.at[pl.ds(s * ROWS_PER_TILE, ROWS_PER_TILE)],
                    out_hbm.at[c, pl.ds(s * ROWS_PER_TILE, ROWS_PER_TILE)])


def _tc_body(eps_ref, x_ref, a0_ref, a1_ref, w_ref, b_ref, o_ref):
    h = (x_ref[...] * (1.0 + eps_ref[0]) + (a0_ref[...] + a1_ref[...])) * SCALE
    acc = jnp.dot(h, w_ref[...], preferred_element_type=jnp.float32)
    o_ref[...] = jnp.maximum(acc + b_ref[...], 0.0)


_BLK = 1000


def _tc_dense(eps, x, a0, a1, W, b2):
    grid = (N_NODES // _BLK,)
    return pl.pallas_call(
        _tc_body,
        grid=grid,
        in_specs=[
            pl.BlockSpec(memory_space=pltpu.SMEM),
            pl.BlockSpec((_BLK, D), lambda i: (i, 0)),
            pl.BlockSpec((_BLK, D), lambda i: (i, 0)),
            pl.BlockSpec((_BLK, D), lambda i: (i, 0)),
            pl.BlockSpec((D, D), lambda i: (0, 0)),
            pl.BlockSpec((1, D), lambda i: (0, 0)),
        ],
        out_specs=pl.BlockSpec((_BLK, D), lambda i: (i, 0)),
        out_shape=jax.ShapeDtypeStruct((N_NODES, D), jnp.float32),
    )(eps, x, a0, a1, W, b2)


def kernel(x, edge_index, eps, W, b):
    src = edge_index[:, 0].reshape(NW * NCHUNK, K)
    dst = edge_index[:, 1].reshape(NW * NCHUNK, K)
    agg = _sc_aggregate(x, src, dst)
    return _tc_dense(eps, x, agg[0], agg[1], W, b.reshape(1, D))


# speedup vs baseline: 6.5750x; 6.5750x over previous
import jax
import jax.numpy as jnp
from jax import lax
from jax.experimental import pallas as pl
from jax.experimental.pallas import tpu as pltpu
from jax.experimental.pallas import tpu_sc as plsc
import functools

N_NODES = 10000
N_EDGES = 320000
D = 128
NC = 2
NS = 16
NW = NC * NS
K = 128
NCHUNK = 78
EPT = NCHUNK * K
EXTRA = N_EDGES - NW * EPT
R0 = 624
SCALE = 1.0 / 33.0

_mesh = plsc.VectorSubcoreMesh(core_axis_name="c", subcore_axis_name="s")


@functools.partial(
    pl.kernel,
    out_type=jax.ShapeDtypeStruct((NC, N_NODES, D), jnp.float32),
    mesh=_mesh,
    scratch_types=[
        pltpu.VMEM((K,), jnp.int32),
        pltpu.VMEM((K,), jnp.int32),
        pltpu.VMEM((K, D), jnp.float32),
        pltpu.VMEM((104, D), jnp.float32),
        pltpu.VMEM_SHARED((N_NODES, D), jnp.float32),
    ],
)
def _sc_aggregate(x_hbm, src_hbm, dst_hbm, out_hbm, src_v, dst_v, rows_v,
                  zbuf, agg_sp):
    c = lax.axis_index("c")
    s = lax.axis_index("s")
    wid = c * NS + s

    def zero_body(i, _):
        zbuf[i // 8, pl.ds((i % 8) * 16, 16)] = jnp.zeros((16,), jnp.float32)
        return 0
    lax.fori_loop(0, 104 * (D // 16), zero_body, 0)

    def zcopy_body(k, _):
        pltpu.sync_copy(zbuf, agg_sp.at[pl.ds(s * R0 + k * 104, 104)])
        return 0
    lax.fori_loop(0, R0 // 104, zcopy_body, 0)

    @pl.when(s == NS - 1)
    def _():
        pltpu.sync_copy(zbuf.at[pl.ds(0, 16)],
                        agg_sp.at[pl.ds(NS * R0, 16)])

    plsc.subcore_barrier()

    def chunk(base):
        pltpu.sync_copy(src_hbm.at[pl.ds(base, K)], src_v)
        pltpu.sync_copy(dst_hbm.at[pl.ds(base, K)], dst_v)
        pltpu.sync_copy(x_hbm.at[src_v], rows_v)
        pltpu.sync_copy(rows_v, agg_sp.at[dst_v], add=True)

    def edge_body(j, _):
        chunk(wid * EPT + j * K)
        return 0
    lax.fori_loop(0, NCHUNK, edge_body, 0)

    @pl.when(wid < EXTRA // K)
    def _():
        chunk(NW * EPT + wid * K)

    plsc.subcore_barrier()

    pltpu.sync_copy(agg_sp.at[pl.ds(s * R0, R0)],
                    out_hbm.at[c, pl.ds(s * R0, R0)])

    @pl.when(s == NS - 1)
    def _():
        pltpu.sync_copy(agg_sp.at[pl.ds(NS * R0, 16)],
                        out_hbm.at[c, pl.ds(NS * R0, 16)])


def _tc_body(eps_ref, x_ref, a0_ref, a1_ref, w_ref, b_ref, o_ref):
    h = (x_ref[...] * (1.0 + eps_ref[0]) + (a0_ref[...] + a1_ref[...])) * SCALE
    acc = jnp.dot(h, w_ref[...], preferred_element_type=jnp.float32)
    o_ref[...] = jnp.maximum(acc + b_ref[...], 0.0)


_BLK = 1000


def _tc_dense(eps, x, a0, a1, W, b2):
    grid = (N_NODES // _BLK,)
    return pl.pallas_call(
        _tc_body,
        grid=grid,
        in_specs=[
            pl.BlockSpec(memory_space=pltpu.SMEM),
            pl.BlockSpec((_BLK, D), lambda i: (i, 0)),
            pl.BlockSpec((_BLK, D), lambda i: (i, 0)),
            pl.BlockSpec((_BLK, D), lambda i: (i, 0)),
            pl.BlockSpec((D, D), lambda i: (0, 0)),
            pl.BlockSpec((1, D), lambda i: (0, 0)),
        ],
        out_specs=pl.BlockSpec((_BLK, D), lambda i: (i, 0)),
        out_shape=jax.ShapeDtypeStruct((N_NODES, D), jnp.float32),
    )(eps, x, a0, a1, W, b2)


def kernel(x, edge_index, eps, W, b):
    src = edge_index[:, 0]
    dst = edge_index[:, 1]
    agg = _sc_aggregate(x, src, dst)
    return _tc_dense(eps, x, agg[0], agg[1], W, b.reshape(1, D))


# speedup vs baseline: 10.9630x; 1.6674x over previous
import jax
import jax.numpy as jnp
from jax import lax
from jax.experimental import pallas as pl
from jax.experimental.pallas import tpu as pltpu
from jax.experimental.pallas import tpu_sc as plsc
import functools

N_NODES = 10000
N_EDGES = 320000
D = 128
NC = 2
NS = 16
NW = NC * NS
K = 128
NCHUNK = 78
EPT = NCHUNK * K
EXTRA = N_EDGES - NW * EPT
R0 = 624
SCALE = 1.0 / 33.0

_mesh = plsc.VectorSubcoreMesh(core_axis_name="c", subcore_axis_name="s")


@functools.partial(
    pl.kernel,
    out_type=jax.ShapeDtypeStruct((NC, N_NODES, D), jnp.float32),
    mesh=_mesh,
    scratch_types=[
        pltpu.VMEM((K,), jnp.int32),
        pltpu.VMEM((K,), jnp.int32),
        pltpu.VMEM((K,), jnp.int32),
        pltpu.VMEM((K,), jnp.int32),
        pltpu.VMEM((K, D), jnp.float32),
        pltpu.VMEM((K, D), jnp.float32),
        pltpu.VMEM((104, D), jnp.float32),
        pltpu.VMEM_SHARED((N_NODES, D), jnp.float32),
        pltpu.SemaphoreType.DMA((2,)),
        pltpu.SemaphoreType.DMA((2,)),
        pltpu.SemaphoreType.DMA((2,)),
        pltpu.SemaphoreType.DMA((2,)),
    ],
)
def _sc_aggregate(x_hbm, src_hbm, dst_hbm, out_hbm, s0, s1, d0, d1, r0, r1,
                  zbuf, agg_sp, si, sd, sr, ss):
    c = lax.axis_index("c")
    s = lax.axis_index("s")
    wid = c * NS + s
    srcs, dsts, rows = (s0, s1), (d0, d1), (r0, r1)

    def zero_body(i, _):
        zbuf[i // 8, pl.ds((i % 8) * 16, 16)] = jnp.zeros((16,), jnp.float32)
        return 0
    lax.fori_loop(0, 104 * (D // 16), zero_body, 0)

    def zcopy_body(k, _):
        pltpu.sync_copy(zbuf, agg_sp.at[pl.ds(s * R0 + k * 104, 104)])
        return 0
    lax.fori_loop(0, R0 // 104, zcopy_body, 0)

    @pl.when(s == NS - 1)
    def _():
        pltpu.sync_copy(zbuf.at[pl.ds(0, 16)], agg_sp.at[pl.ds(NS * R0, 16)])

    plsc.subcore_barrier()

    n_tile = NCHUNK + (wid < EXTRA // K).astype(jnp.int32)

    def base(jj):
        return jnp.where(jj < NCHUNK, wid * EPT + jj * K, NW * EPT + wid * K)

    def start_idx(b, jj):
        off = base(jj)
        pltpu.async_copy(src_hbm.at[pl.ds(off, K)], srcs[b], si.at[b])
        pltpu.async_copy(dst_hbm.at[pl.ds(off, K)], dsts[b], sd.at[b])

    def wait_idx(b):
        pltpu.make_async_copy(src_hbm.at[pl.ds(0, K)], srcs[b], si.at[b]).wait()
        pltpu.make_async_copy(dst_hbm.at[pl.ds(0, K)], dsts[b], sd.at[b]).wait()

    def start_gather(b):
        pltpu.async_copy(x_hbm.at[srcs[b]], rows[b], sr.at[b])

    def wait_gather(b):
        pltpu.make_async_copy(x_hbm.at[srcs[b]], rows[b], sr.at[b]).wait()

    def start_scatter(b):
        pltpu.async_copy(rows[b], agg_sp.at[dsts[b]], ss.at[b], add=True)

    def wait_scatter(b):
        pltpu.make_async_copy(rows[b], agg_sp.at[dsts[b]], ss.at[b]).wait()

    start_idx(0, 0)
    wait_idx(0)
    start_gather(0)
    start_idx(1, 1)

    def step(jj, b):
        nb = 1 - b

        @pl.when(jj < n_tile)
        def _():
            wait_gather(b)

            @pl.when(jj + 1 < n_tile)
            def _():
                wait_idx(nb)

                @pl.when(jj >= 1)
                def _():
                    wait_scatter(nb)
                start_gather(nb)

            start_scatter(b)

            @pl.when(jj + 2 < n_tile)
            def _():
                start_idx(b, jj + 2)

    def pair_body(i, _):
        step(2 * i, 0)
        step(2 * i + 1, 1)
        return 0
    lax.fori_loop(0, (NCHUNK + 2) // 2, pair_body, 0)

    wait_scatter(0)
    wait_scatter(1)

    plsc.subcore_barrier()

    pltpu.sync_copy(agg_sp.at[pl.ds(s * R0, R0)],
                    out_hbm.at[c, pl.ds(s * R0, R0)])

    @pl.when(s == NS - 1)
    def _():
        pltpu.sync_copy(agg_sp.at[pl.ds(NS * R0, 16)],
                        out_hbm.at[c, pl.ds(NS * R0, 16)])


def _tc_body(eps_ref, x_ref, a0_ref, a1_ref, w_ref, b_ref, o_ref):
    h = (x_ref[...] * (1.0 + eps_ref[0]) + (a0_ref[...] + a1_ref[...])) * SCALE
    acc = jnp.dot(h, w_ref[...], preferred_element_type=jnp.float32)
    o_ref[...] = jnp.maximum(acc + b_ref[...], 0.0)


_BLK = 1000


def _tc_dense(eps, x, a0, a1, W, b2):
    grid = (N_NODES // _BLK,)
    return pl.pallas_call(
        _tc_body,
        grid=grid,
        in_specs=[
            pl.BlockSpec(memory_space=pltpu.SMEM),
            pl.BlockSpec((_BLK, D), lambda i: (i, 0)),
            pl.BlockSpec((_BLK, D), lambda i: (i, 0)),
            pl.BlockSpec((_BLK, D), lambda i: (i, 0)),
            pl.BlockSpec((D, D), lambda i: (0, 0)),
            pl.BlockSpec((1, D), lambda i: (0, 0)),
        ],
        out_specs=pl.BlockSpec((_BLK, D), lambda i: (i, 0)),
        out_shape=jax.ShapeDtypeStruct((N_NODES, D), jnp.float32),
    )(eps, x, a0, a1, W, b2)


def kernel(x, edge_index, eps, W, b):
    src = edge_index[:, 0]
    dst = edge_index[:, 1]
    agg = _sc_aggregate(x, src, dst)
    return _tc_dense(eps, x, agg[0], agg[1], W, b.reshape(1, D))


# speedup vs baseline: 12.9899x; 1.1849x over previous
import jax
import jax.numpy as jnp
from jax import lax
from jax.experimental import pallas as pl
from jax.experimental.pallas import tpu as pltpu
from jax.experimental.pallas import tpu_sc as plsc
import functools

N_NODES = 10000
N_EDGES = 320000
D = 128
NC = 2
NS = 16
NW = NC * NS
K = 104
NCHUNK = 96
EPT = NCHUNK * K
KE = 64
EXTRA = N_EDGES - NW * EPT
R0 = 624
SCALE = 1.0 / 33.0

_mesh = plsc.VectorSubcoreMesh(core_axis_name="c", subcore_axis_name="s")


@functools.partial(
    pl.kernel,
    out_type=jax.ShapeDtypeStruct((NC, N_NODES, D), jnp.float32),
    mesh=_mesh,
    scratch_types=[
        pltpu.VMEM((K,), jnp.int32),
        pltpu.VMEM((K,), jnp.int32),
        pltpu.VMEM((K,), jnp.int32),
        pltpu.VMEM((K,), jnp.int32),
        pltpu.VMEM((K,), jnp.int32),
        pltpu.VMEM((K,), jnp.int32),
        pltpu.VMEM((K, D), jnp.float32),
        pltpu.VMEM((K, D), jnp.float32),
        pltpu.VMEM((K, D), jnp.float32),
        pltpu.VMEM((KE,), jnp.int32),
        pltpu.VMEM((KE,), jnp.int32),
        pltpu.VMEM_SHARED((N_NODES, D), jnp.float32),
        pltpu.SemaphoreType.DMA((3,)),
        pltpu.SemaphoreType.DMA((3,)),
        pltpu.SemaphoreType.DMA((3,)),
        pltpu.SemaphoreType.DMA((3,)),
    ],
)
def _sc_aggregate(x_hbm, src_hbm, dst_hbm, out_hbm, s0, s1, s2, d0, d1, d2,
                  r0, r1, r2, se, de, agg_sp, si, sd, sr, ss):
    c = lax.axis_index("c")
    s = lax.axis_index("s")
    wid = c * NS + s
    srcs, dsts, rows = (s0, s1, s2), (d0, d1, d2), (r0, r1, r2)

    def zero_body(i, _):
        r0[i // 8, pl.ds((i % 8) * 16, 16)] = jnp.zeros((16,), jnp.float32)
        return 0
    lax.fori_loop(0, K * (D // 16), zero_body, 0)

    def zcopy_body(k, _):
        pltpu.sync_copy(r0, agg_sp.at[pl.ds(s * R0 + k * K, K)])
        return 0
    lax.fori_loop(0, R0 // K, zcopy_body, 0)

    @pl.when(s == NS - 1)
    def _():
        pltpu.sync_copy(r0.at[pl.ds(0, 16)], agg_sp.at[pl.ds(NS * R0, 16)])

    plsc.subcore_barrier()

    def start_idx(b, jj):
        off = wid * EPT + jj * K
        pltpu.async_copy(src_hbm.at[pl.ds(off, K)], srcs[b], si.at[b])
        pltpu.async_copy(dst_hbm.at[pl.ds(off, K)], dsts[b], sd.at[b])

    def wait_idx(b):
        pltpu.make_async_copy(src_hbm.at[pl.ds(0, K)], srcs[b], si.at[b]).wait()
        pltpu.make_async_copy(dst_hbm.at[pl.ds(0, K)], dsts[b], sd.at[b]).wait()

    def start_gather(b):
        pltpu.async_copy(x_hbm.at[srcs[b]], rows[b], sr.at[b])

    def wait_gather(b):
        pltpu.make_async_copy(x_hbm.at[srcs[b]], rows[b], sr.at[b]).wait()

    def start_scatter(b):
        pltpu.async_copy(rows[b], agg_sp.at[dsts[b]], ss.at[b], add=True)

    def wait_scatter(b):
        pltpu.make_async_copy(rows[b], agg_sp.at[dsts[b]], ss.at[b]).wait()

    start_idx(0, 0)
    start_idx(1, 1)
    start_idx(2, 2)
    wait_idx(0)
    start_gather(0)
    wait_idx(1)
    start_gather(1)

    def step(jj, b):
        b2 = (b + 2) % 3
        wait_gather(b)
        start_scatter(b)

        @pl.when(jj + 3 < NCHUNK)
        def _():
            start_idx(b, jj + 3)

        @pl.when(jj + 2 < NCHUNK)
        def _():
            wait_idx(b2)

            @pl.when(jj >= 1)
            def _():
                wait_scatter(b2)
            start_gather(b2)

    def trio_body(i, _):
        step(3 * i, 0)
        step(3 * i + 1, 1)
        step(3 * i + 2, 2)
        return 0
    lax.fori_loop(0, NCHUNK // 3, trio_body, 0)

    wait_scatter(0)
    wait_scatter(1)
    wait_scatter(2)

    @pl.when(wid < EXTRA // KE)
    def _():
        off = NW * EPT + wid * KE
        pltpu.sync_copy(src_hbm.at[pl.ds(off, KE)], se)
        pltpu.sync_copy(dst_hbm.at[pl.ds(off, KE)], de)
        pltpu.sync_copy(x_hbm.at[se], r0.at[pl.ds(0, KE)])
        pltpu.sync_copy(r0.at[pl.ds(0, KE)], agg_sp.at[de], add=True)

    plsc.subcore_barrier()

    pltpu.sync_copy(agg_sp.at[pl.ds(s * R0, R0)],
                    out_hbm.at[c, pl.ds(s * R0, R0)])

    @pl.when(s == NS - 1)
    def _():
        pltpu.sync_copy(agg_sp.at[pl.ds(NS * R0, 16)],
                        out_hbm.at[c, pl.ds(NS * R0, 16)])


def _tc_body(eps_ref, x_ref, a0_ref, a1_ref, w_ref, b_ref, o_ref):
    h = (x_ref[...] * (1.0 + eps_ref[0]) + (a0_ref[...] + a1_ref[...])) * SCALE
    acc = jnp.dot(h, w_ref[...], preferred_element_type=jnp.float32)
    o_ref[...] = jnp.maximum(acc + b_ref[...], 0.0)


_BLK = 1000


def _tc_dense(eps, x, a0, a1, W, b2):
    grid = (N_NODES // _BLK,)
    return pl.pallas_call(
        _tc_body,
        grid=grid,
        in_specs=[
            pl.BlockSpec(memory_space=pltpu.SMEM),
            pl.BlockSpec((_BLK, D), lambda i: (i, 0)),
            pl.BlockSpec((_BLK, D), lambda i: (i, 0)),
            pl.BlockSpec((_BLK, D), lambda i: (i, 0)),
            pl.BlockSpec((D, D), lambda i: (0, 0)),
            pl.BlockSpec((1, D), lambda i: (0, 0)),
        ],
        out_specs=pl.BlockSpec((_BLK, D), lambda i: (i, 0)),
        out_shape=jax.ShapeDtypeStruct((N_NODES, D), jnp.float32),
    )(eps, x, a0, a1, W, b2)


def kernel(x, edge_index, eps, W, b):
    src = edge_index[:, 0]
    dst = edge_index[:, 1]
    agg = _sc_aggregate(x, src, dst)
    return _tc_dense(eps, x, agg[0], agg[1], W, b.reshape(1, D))


# speedup vs baseline: 13.6155x; 1.0482x over previous
import jax
import jax.numpy as jnp
from jax import lax
from jax.experimental import pallas as pl
from jax.experimental.pallas import tpu as pltpu
from jax.experimental.pallas import tpu_sc as plsc
import functools

N_NODES = 10000
N_EDGES = 320000
D = 128
NC = 2
NS = 16
NW = NC * NS
K = 104
NCHUNK = 96
EPT = NCHUNK * K
KE = 64
EXTRA = N_EDGES - NW * EPT
R0 = 624
SCALE = 1.0 / 33.0

_mesh = plsc.VectorSubcoreMesh(core_axis_name="c", subcore_axis_name="s")


@functools.partial(
    pl.kernel,
    out_type=jax.ShapeDtypeStruct((NC, N_NODES, D), jnp.float32),
    mesh=_mesh,
    scratch_types=[
        pltpu.VMEM((K,), jnp.int32),
        pltpu.VMEM((K,), jnp.int32),
        pltpu.VMEM((K,), jnp.int32),
        pltpu.VMEM((K,), jnp.int32),
        pltpu.VMEM((K,), jnp.int32),
        pltpu.VMEM((K,), jnp.int32),
        pltpu.VMEM((K, D), jnp.float32),
        pltpu.VMEM((K, D), jnp.float32),
        pltpu.VMEM((K, D), jnp.float32),
        pltpu.VMEM((KE,), jnp.int32),
        pltpu.VMEM((KE,), jnp.int32),
        pltpu.VMEM_SHARED((N_NODES, D), jnp.float32),
        pltpu.SemaphoreType.DMA((3,)),
        pltpu.SemaphoreType.DMA((3,)),
        pltpu.SemaphoreType.DMA((3,)),
        pltpu.SemaphoreType.DMA((3,)),
    ],
)
def _sc_aggregate(x_hbm, src_hbm, dst_hbm, out_hbm, s0, s1, s2, d0, d1, d2,
                  r0, r1, r2, se, de, agg_sp, si, sd, sr, ss):
    c = lax.axis_index("c")
    s = lax.axis_index("s")
    wid = c * NS + s
    srcs, dsts, rows = (s0, s1, s2), (d0, d1, d2), (r0, r1, r2)

    def zero_body(i, _):
        r0[i // 8, pl.ds((i % 8) * 16, 16)] = jnp.zeros((16,), jnp.float32)
        return 0
    lax.fori_loop(0, K * (D // 16), zero_body, 0)

    def zcopy_body(k, _):
        pltpu.sync_copy(r0, agg_sp.at[pl.ds(s * R0 + k * K, K)])
        return 0
    lax.fori_loop(0, R0 // K, zcopy_body, 0)

    @pl.when(s == NS - 1)
    def _():
        pltpu.sync_copy(r0.at[pl.ds(0, 16)], agg_sp.at[pl.ds(NS * R0, 16)])

    plsc.subcore_barrier()

    def start_idx(b, jj):
        off = wid * EPT + jj * K
        pltpu.async_copy(src_hbm.at[pl.ds(off, K)], srcs[b], si.at[b])
        pltpu.async_copy(dst_hbm.at[pl.ds(off, K)], dsts[b], sd.at[b])

    def wait_idx(b):
        pltpu.make_async_copy(src_hbm.at[pl.ds(0, K)], srcs[b], si.at[b]).wait()
        pltpu.make_async_copy(dst_hbm.at[pl.ds(0, K)], dsts[b], sd.at[b]).wait()

    def start_gather(b):
        pltpu.async_copy(x_hbm.at[srcs[b]], rows[b], sr.at[b])

    def wait_gather(b):
        pltpu.make_async_copy(x_hbm.at[srcs[b]], rows[b], sr.at[b]).wait()

    def start_scatter(b):
        pltpu.async_copy(rows[b], agg_sp.at[dsts[b]], ss.at[b], add=True)

    def wait_scatter(b):
        pltpu.make_async_copy(rows[b], agg_sp.at[dsts[b]], ss.at[b]).wait()

    start_idx(0, 0)
    start_idx(1, 1)
    start_idx(2, 2)
    wait_idx(0)
    start_gather(0)
    wait_idx(1)
    start_gather(1)

    def step(jj, b):
        b2 = (b + 2) % 3
        wait_gather(b)
        start_scatter(b)

        @pl.when(jj + 3 < NCHUNK)
        def _():
            start_idx(b, jj + 3)

        @pl.when(jj + 2 < NCHUNK)
        def _():
            wait_idx(b2)

            @pl.when(jj >= 1)
            def _():
                wait_scatter(b2)
            start_gather(b2)

    def trio_body(i, _):
        step(3 * i, 0)
        step(3 * i + 1, 1)
        step(3 * i + 2, 2)
        return 0
    lax.fori_loop(0, NCHUNK // 3, trio_body, 0)

    wait_scatter(0)
    wait_scatter(1)
    wait_scatter(2)

    @pl.when(wid < EXTRA // KE)
    def _():
        off = NW * EPT + wid * KE
        pltpu.sync_copy(src_hbm.at[pl.ds(off, KE)], se)
        pltpu.sync_copy(dst_hbm.at[pl.ds(off, KE)], de)
        pltpu.sync_copy(x_hbm.at[se], r0.at[pl.ds(0, KE)])
        pltpu.sync_copy(r0.at[pl.ds(0, KE)], agg_sp.at[de], add=True)

    plsc.subcore_barrier()

    pltpu.sync_copy(agg_sp.at[pl.ds(s * R0, R0)],
                    out_hbm.at[c, pl.ds(s * R0, R0)])

    @pl.when(s == NS - 1)
    def _():
        pltpu.sync_copy(agg_sp.at[pl.ds(NS * R0, 16)],
                        out_hbm.at[c, pl.ds(NS * R0, 16)])


def _tc_body(eps_ref, x_ref, a0_ref, a1_ref, w_ref, b_ref, o_ref):
    h = (x_ref[...] * (1.0 + eps_ref[0]) + (a0_ref[0] + a1_ref[0])) * SCALE
    acc = jnp.dot(h, w_ref[...], preferred_element_type=jnp.float32)
    o_ref[...] = jnp.maximum(acc + b_ref[...], 0.0)


_BLK = 1000


def _tc_dense(eps, x, a0, a1, W, b2):
    grid = (N_NODES // _BLK,)
    return pl.pallas_call(
        _tc_body,
        grid=grid,
        in_specs=[
            pl.BlockSpec(memory_space=pltpu.SMEM),
            pl.BlockSpec((_BLK, D), lambda i: (i, 0)),
            pl.BlockSpec((1, _BLK, D), lambda i: (0, i, 0)),
            pl.BlockSpec((1, _BLK, D), lambda i: (1, i, 0)),
            pl.BlockSpec((D, D), lambda i: (0, 0)),
            pl.BlockSpec((1, D), lambda i: (0, 0)),
        ],
        out_specs=pl.BlockSpec((_BLK, D), lambda i: (i, 0)),
        out_shape=jax.ShapeDtypeStruct((N_NODES, D), jnp.float32),
    )(eps, x, a0, a1, W, b2)


def kernel(x, edge_index, eps, W, b):
    src = edge_index[:, 0]
    dst = edge_index[:, 1]
    agg = _sc_aggregate(x, src, dst)
    return _tc_dense(eps, x, agg, agg, W, b.reshape(1, D))
